# Initial kernel scaffold; baseline (speedup 1.0000x reference)
import jax
import jax.numpy as jnp
from jax.experimental import pallas as pl


def kernel(x, edge_index, W1, b1, W2, b2, W3, b3):
    raise NotImplementedError("write your pallas kernel here")


# speedup vs baseline: 9.3531x; 9.3531x over previous
import functools

import jax
import jax.numpy as jnp
from jax import lax
from jax.experimental import pallas as pl
from jax.experimental.pallas import tpu as pltpu
from jax.experimental.pallas import tpu_sc as plsc

N = 10000
E = 160000
D = 256
F = 128
NS = 16
NC = 2
CH = 80
NCH = E // NS // CH
EPW = E // (NC * NS)
NHIST = 8
NPAD = 10240
RPT = 1000
NT_IO = N // RPT
BN = 1000
NBI = N // BN

@functools.cache
def _sc_mesh():
    return plsc.VectorSubcoreMesh(
        core_axis_name="c", subcore_axis_name="s", num_cores=NC, num_subcores=NS)


def _deg_body(dst3, zeros, ones, out, cnt_sp, dstv, onesv):
    c = lax.axis_index("c")
    s = lax.axis_index("s")
    stripe = NPAD // NS
    pltpu.sync_copy(dst3.at[s], dstv)
    pltpu.sync_copy(ones, onesv)
    pltpu.sync_copy(zeros, cnt_sp.at[pl.ds(s * stripe, stripe)])
    plsc.subcore_barrier()

    half = (NCH + 1) // 2
    base = c * half

    def chunk(t, v):
        pltpu.sync_copy(onesv, cnt_sp.at[dstv.at[base + t]], add=True)
        return v

    lax.fori_loop(0, half - c, chunk, 0)
    plsc.subcore_barrier()
    pltpu.sync_copy(cnt_sp.at[pl.ds(s * stripe, stripe)],
                    out.at[c, pl.ds(s * stripe, stripe)])


@functools.cache
def _deg_kernel():
    return pl.kernel(
        _deg_body,
        out_type=jax.ShapeDtypeStruct((NC, NPAD, 128), jnp.float32),
        mesh=_sc_mesh(),
        scratch_types=[
            pltpu.VMEM_SHARED((NPAD, 128), jnp.float32),
            pltpu.VMEM((NCH, CH), jnp.int32),
            pltpu.VMEM((CH, 128), jnp.float32),
        ],
    )


def _prop_body(g, srcp, dst, out, acc_sp, srcv, dstv, rows, sem):
    c = lax.axis_index("c")
    s = lax.axis_index("s")
    r0 = s * RPT
    pltpu.sync_copy(srcp.at[c, s], srcv)
    pltpu.sync_copy(dst.at[s], dstv)

    @pl.when(s < NT_IO)
    def _():
        pltpu.sync_copy(g.at[pl.ds(c * N + r0, RPT)], acc_sp.at[pl.ds(r0, RPT)])

    plsc.subcore_barrier()

    def chunk(t, v):
        pltpu.async_copy(g.at[srcv.at[t]], rows, sem).wait()
        pltpu.sync_copy(rows, acc_sp.at[dstv.at[t]], add=True)
        return v

    lax.fori_loop(0, NCH, chunk, 0)
    plsc.subcore_barrier()

    @pl.when(s < NT_IO)
    def _():
        pltpu.sync_copy(acc_sp.at[pl.ds(r0, RPT)], out.at[pl.ds(c * N + r0, RPT)])


@functools.cache
def _prop_kernel():
    return pl.kernel(
        _prop_body,
        out_type=jax.ShapeDtypeStruct((NC * N, F), jnp.float32),
        mesh=_sc_mesh(),
        scratch_types=[
            pltpu.VMEM_SHARED((N, F), jnp.float32),
            pltpu.VMEM((NCH, CH), jnp.int32),
            pltpu.VMEM((NCH, CH), jnp.int32),
            pltpu.VMEM((CH, F), jnp.float32),
            pltpu.SemaphoreType.DMA,
        ],
    )


def _dis(deg_ref):
    return lax.rsqrt(deg_ref[0][:, 0:1] + deg_ref[1][:, 0:1] + 1.0)


def _mm_first_body(x_ref, w_ref, deg_ref, o_ref):
    k = pl.program_id(2)
    part = jnp.dot(x_ref[...], w_ref[...], preferred_element_type=jnp.float32)

    @pl.when(k == 0)
    def _():
        o_ref[...] = part

    @pl.when(k == 1)
    def _():
        o_ref[...] = (o_ref[...] + part) * _dis(deg_ref)


def _mm_mid_body(a_ref, w_ref, deg_ref, b_ref, o_ref):
    k = pl.program_id(2)
    dis = _dis(deg_ref)
    z = jnp.maximum(a_ref[...] * dis + b_ref[0], 0.0)
    part = jnp.dot(z, w_ref[...], preferred_element_type=jnp.float32)

    @pl.when(k == 0)
    def _():
        o_ref[...] = part

    @pl.when(k == 1)
    def _():
        o_ref[...] = (o_ref[...] + part) * dis


def _final_body(a_ref, deg_ref, b_ref, o_ref):
    o_ref[...] = a_ref[...] * _dis(deg_ref) + b_ref[...]


def _mm_first(x, w, degp):
    return pl.pallas_call(
        _mm_first_body,
        grid=(NBI, 2, 2),
        in_specs=[
            pl.BlockSpec((BN, 128), lambda i, j, k: (i, k)),
            pl.BlockSpec((128, 128), lambda i, j, k: (k, j)),
            pl.BlockSpec((NC, BN, 128), lambda i, j, k: (0, i, 0)),
        ],
        out_specs=pl.BlockSpec((BN, F), lambda i, j, k: (j * NBI + i, 0)),
        out_shape=jax.ShapeDtypeStruct((NC * N, F), jnp.float32),
    )(x, w, degp)


def _mm_mid(a, w, degp, b):
    return pl.pallas_call(
        _mm_mid_body,
        grid=(NBI, 2, 2),
        in_specs=[
            pl.BlockSpec((BN, F), lambda i, j, k: (k * NBI + i, 0)),
            pl.BlockSpec((128, 128), lambda i, j, k: (k, j)),
            pl.BlockSpec((NC, BN, 128), lambda i, j, k: (0, i, 0)),
            pl.BlockSpec((1, 1, 128), lambda i, j, k: (k, 0, 0)),
        ],
        out_specs=pl.BlockSpec((BN, F), lambda i, j, k: (j * NBI + i, 0)),
        out_shape=jax.ShapeDtypeStruct((NC * N, F), jnp.float32),
    )(a, w, degp, b)


def _final(a, degp, b):
    return pl.pallas_call(
        _final_body,
        grid=(NBI, 2),
        in_specs=[
            pl.BlockSpec((BN, F), lambda i, j: (j * NBI + i, 0)),
            pl.BlockSpec((NC, BN, 128), lambda i, j: (0, i, 0)),
            pl.BlockSpec((1, 128), lambda i, j: (0, j)),
        ],
        out_specs=pl.BlockSpec((BN, 128), lambda i, j: (i, j)),
        out_shape=jax.ShapeDtypeStruct((N, D), jnp.float32),
    )(a, degp, b)


def kernel(x, edge_index, W1, b1, W2, b2, W3, b3):
    src = edge_index[0]
    dst = edge_index[1]
    s3 = src.reshape(NS, NCH, CH)
    srcp = jnp.stack([s3, s3 + N])
    dst3 = dst.reshape(NS, NCH, CH)
    zeros = jnp.zeros((NPAD // NS, 128), jnp.float32)
    ones = jnp.ones((CH, 128), jnp.float32)

    degp = _deg_kernel()(dst3, zeros, ones)

    g = _mm_first(x, W1, degp)
    acc = _prop_kernel()(g, srcp, dst3)
    g = _mm_mid(acc, W2, degp, b1.reshape(NC, 1, 128))
    acc = _prop_kernel()(g, srcp, dst3)
    g = _mm_mid(acc, W3, degp, b2.reshape(NC, 1, 128))
    acc = _prop_kernel()(g, srcp, dst3)
    return _final(acc, degp, b3.reshape(1, D))
